# baseline (speedup 1.0000x reference)
import numpy as np
import jax
import jax.numpy as jnp
from jax import lax
from jax.experimental import pallas as pl
from jax.experimental.pallas import tpu as pltpu
from jax.experimental.pallas import tpu_sc as plsc

_B = 524288
_L = 16
_C = 2
_NC, _NS = 2, 16
_NW = _NC * _NS
_CH = 128
_PTS = _B // _NW
_NCHUNK = _PTS // _CH
_GSZ = 8 * _C * _CH
_P1 = np.int64(2654435761).astype(np.int32)
_P2 = np.int64(805459861).astype(np.int32)


def _build_tab():
    rows = np.zeros((_L, 6), np.int64)
    offset = 0
    for l in range(_L):
        scale = 2.0 ** l * 16.0 - 1.0
        res = int(np.ceil(scale)) + 1
        params = min(2 ** 19, res ** 3)
        params = int(np.ceil(params / 8) * 8)
        is_hash = 1 if res ** 3 > params else 0
        rows[l, 1] = 0 if is_hash else res
        rows[l, 2] = 0 if is_hash else res * res
        rows[l, 3] = params - 1
        rows[l, 4] = 2 * offset
        rows[l, 5] = is_hash
        offset += params
    tab = np.repeat(rows.astype(np.int32)[:, :, None], 16, axis=2)
    scales = np.array([2.0 ** l * 16.0 - 1.0 for l in range(_L)], np.float32)
    ftab = np.repeat(scales[:, None], 16, axis=1)
    return tab.reshape(-1), ftab.reshape(-1)


_TAB, _FTAB = _build_tab()


def _sc_body(inp_hbm, tab_hbm, ftab_hbm, emb_hbm, out_hbm,
             tab_v, ftab_v, xv, yv, zv,
             fb0, fb1, ib0, ib1, gb0, gb1, ov, sem0, sem1):
    wid = lax.axis_index("s") * _NC + lax.axis_index("c")
    pltpu.sync_copy(tab_hbm, tab_v)
    pltpu.sync_copy(ftab_hbm, ftab_v)

    _HMASK = (1 << 19) - 1

    def phase1h(l, fb, ib):
        scale = ftab_v[pl.ds(l * 16, 16)]
        offv2 = tab_v[pl.ds(l * 96 + 64, 16)]
        for g in range(_CH // 16):
            s = pl.ds(g * 16, 16)
            px = xv[s] * scale + 0.5
            py = yv[s] * scale + 0.5
            pz = zv[s] * scale + 0.5
            gx = px.astype(jnp.int32)
            gy = py.astype(jnp.int32)
            gz = pz.astype(jnp.int32)
            fb[pl.ds(g * 16, 16)] = px - gx.astype(jnp.float32)
            fb[pl.ds(_CH + g * 16, 16)] = py - gy.astype(jnp.float32)
            fb[pl.ds(2 * _CH + g * 16, 16)] = pz - gz.astype(jnp.float32)
            x1 = gx + 1
            h1a = gy * _P1
            h1b = h1a + _P1
            h2a = gz * _P2
            h2b = h2a + _P2
            for c in range(8):
                s0 = x1 if (c & 1) else gx
                w = s0 ^ (h1b if (c & 2) else h1a) ^ (h2b if (c & 4) else h2a)
                a0 = ((w & _HMASK) * 2) - (w & 127) + offv2
                cb = c * 2 * _CH + g * 32
                ib[pl.ds(cb, 16)] = a0
                ib[pl.ds(cb + 16, 16)] = a0 + 128

    def phase1d(l, fb, ib):
        scale = float(2.0 ** l * 16.0 - 1.0)
        res = 16 << l
        mask = res ** 3 - 1
        off2 = 2 * sum((16 << j) ** 3 for j in range(l))
        for g in range(_CH // 16):
            s = pl.ds(g * 16, 16)
            px = xv[s] * scale + 0.5
            py = yv[s] * scale + 0.5
            pz = zv[s] * scale + 0.5
            gx = px.astype(jnp.int32)
            gy = py.astype(jnp.int32)
            gz = pz.astype(jnp.int32)
            fb[pl.ds(g * 16, 16)] = px - gx.astype(jnp.float32)
            fb[pl.ds(_CH + g * 16, 16)] = py - gy.astype(jnp.float32)
            fb[pl.ds(2 * _CH + g * 16, 16)] = pz - gz.astype(jnp.float32)
            x1 = gx + 1
            d1a = gy * res
            d1b = d1a + res
            d2a = gz * (res * res)
            d2b = d2a + (res * res)
            for c in range(8):
                s0 = x1 if (c & 1) else gx
                w = s0 + (d1b if (c & 2) else d1a) + (d2b if (c & 4) else d2a)
                a0 = ((w & mask) * 2) - (w & 127) + off2
                cb = c * 2 * _CH + g * 32
                ib[pl.ds(cb, 16)] = a0
                ib[pl.ds(cb + 16, 16)] = a0 + 128

    def fire(ib, gb, sem):
        pltpu.async_copy(emb_hbm.at[ib], gb, sem)

    def drain(ib, gb, sem):
        pltpu.make_async_copy(emb_hbm.at[ib], gb, sem).wait()

    def phase3(l, fb, gb):
        lvl_off = l * (2 * _CH)
        for g in range(_CH // 16):
            s = pl.ds(g * 16, 16)
            fx = fb[s]
            fy = fb[pl.ds(_CH + g * 16, 16)]
            fz = fb[pl.ds(2 * _CH + g * 16, 16)]
            for ch in range(2):
                v = [gb[pl.ds(c * 2 * _CH + g * 32 + ch * 16, 16)]
                     for c in range(8)]
                u0 = v[0] + fx * (v[1] - v[0])
                u1 = v[2] + fx * (v[3] - v[2])
                u2 = v[4] + fx * (v[5] - v[4])
                u3 = v[6] + fx * (v[7] - v[6])
                w0 = u0 + fy * (u1 - u0)
                w1 = u2 + fy * (u3 - u2)
                r = w0 + fz * (w1 - w0)
                ov[pl.ds(lvl_off + (ch * _CH + g * 16), 16)] = r

    def chunk_body(t, _):
        base = wid * _PTS + t * _CH
        pltpu.sync_copy(inp_hbm.at[pl.ds(base, _CH)], xv)
        pltpu.sync_copy(inp_hbm.at[pl.ds(_B + base, _CH)], yv)
        pltpu.sync_copy(inp_hbm.at[pl.ds(2 * _B + base, _CH)], zv)

        phase1d(0, fb0, ib0)
        fire(ib0, gb0, sem0)
        phase1d(1, fb1, ib1)
        fire(ib1, gb1, sem1)
        drain(ib0, gb0, sem0)
        phase3(jnp.int32(0), fb0, gb0)
        phase1d(2, fb0, ib0)
        fire(ib0, gb0, sem0)
        drain(ib1, gb1, sem1)
        phase3(jnp.int32(1), fb1, gb1)
        phase1h(jnp.int32(3), fb1, ib1)
        fire(ib1, gb1, sem1)
        drain(ib0, gb0, sem0)
        phase3(jnp.int32(2), fb0, gb0)

        def pair_body(k, _):
            la = 2 * k + 4
            phase1h(la, fb0, ib0)
            fire(ib0, gb0, sem0)
            drain(ib1, gb1, sem1)
            phase3(la - 1, fb1, gb1)
            phase1h(la + 1, fb1, ib1)
            fire(ib1, gb1, sem1)
            drain(ib0, gb0, sem0)
            phase3(la, fb0, gb0)

        lax.fori_loop(0, 6, pair_body, None)
        drain(ib1, gb1, sem1)
        phase3(jnp.int32(_L - 1), fb1, gb1)

        pltpu.sync_copy(ov, out_hbm.at[pl.ds(base * 32, _CH * 32)])

    lax.fori_loop(0, _NCHUNK, chunk_body, None)


_mesh = plsc.VectorSubcoreMesh(core_axis_name="c", subcore_axis_name="s",
                               num_cores=_NC, num_subcores=_NS)

_sc_call = pl.kernel(
    _sc_body,
    out_type=jax.ShapeDtypeStruct((_B * _L * _C,), jnp.float32),
    mesh=_mesh,
    scratch_types=[
        pltpu.VMEM((_L * 6 * 16,), jnp.int32),
        pltpu.VMEM((_L * 16,), jnp.float32),
        pltpu.VMEM((_CH,), jnp.float32),
        pltpu.VMEM((_CH,), jnp.float32),
        pltpu.VMEM((_CH,), jnp.float32),
        pltpu.VMEM((3 * _CH,), jnp.float32),
        pltpu.VMEM((3 * _CH,), jnp.float32),
        pltpu.VMEM((_GSZ,), jnp.int32),
        pltpu.VMEM((_GSZ,), jnp.int32),
        pltpu.VMEM((_GSZ,), jnp.float32),
        pltpu.VMEM((_GSZ,), jnp.float32),
        pltpu.VMEM((_CH * _L * _C,), jnp.float32),
        pltpu.SemaphoreType.DMA,
        pltpu.SemaphoreType.DMA,
    ],
)


def _tr_body(x_ref, o_ref):
    o_ref[...] = jnp.transpose(x_ref[...], (0, 2, 1))


_TRB = 16

_tr_call = pl.pallas_call(
    _tr_body,
    grid=(_B // _CH // _TRB,),
    in_specs=[pl.BlockSpec((_TRB, _L * _C, _CH), lambda i: (i, 0, 0))],
    out_specs=pl.BlockSpec((_TRB, _CH, _L * _C), lambda i: (i, 0, 0)),
    out_shape=jax.ShapeDtypeStruct((_B // _CH, _CH, _L * _C), jnp.float32),
)


def kernel(inputs, embeddings):
    inp_t = inputs.T.reshape(-1)
    tab = jnp.asarray(_TAB)
    ftab = jnp.asarray(_FTAB)
    vemb = embeddings.reshape(-1, 128, _C).transpose(0, 2, 1).reshape(-1)
    out = _sc_call(inp_t, tab, ftab, vemb)
    out = _tr_call(out.reshape(_B // _CH, _L * _C, _CH))
    return out.reshape(_B, _L * _C)


# speedup vs baseline: 1.0563x; 1.0563x over previous
import numpy as np
import jax
import jax.numpy as jnp
from jax import lax
from jax.experimental import pallas as pl
from jax.experimental.pallas import tpu as pltpu
from jax.experimental.pallas import tpu_sc as plsc

_B = 524288
_L = 16
_C = 2
_NC, _NS = 2, 16
_NW = _NC * _NS
_CH = 128
_PTS = _B // _NW
_NCHUNK = _PTS // _CH
_GSZ = 8 * _C * _CH
_P1 = np.int64(2654435761).astype(np.int32)
_P2 = np.int64(805459861).astype(np.int32)


def _build_tab():
    rows = np.zeros((_L, 6), np.int64)
    offset = 0
    for l in range(_L):
        scale = 2.0 ** l * 16.0 - 1.0
        res = int(np.ceil(scale)) + 1
        params = min(2 ** 19, res ** 3)
        params = int(np.ceil(params / 8) * 8)
        is_hash = 1 if res ** 3 > params else 0
        rows[l, 1] = 0 if is_hash else res
        rows[l, 2] = 0 if is_hash else res * res
        rows[l, 3] = params - 1
        rows[l, 4] = 2 * offset
        rows[l, 5] = is_hash
        offset += params
    tab = np.repeat(rows.astype(np.int32)[:, :, None], 16, axis=2)
    scales = np.array([2.0 ** l * 16.0 - 1.0 for l in range(_L)], np.float32)
    ftab = np.repeat(scales[:, None], 16, axis=1)
    return tab.reshape(-1), ftab.reshape(-1)


_TAB, _FTAB = _build_tab()


def _sc_body(inp_hbm, tab_hbm, ftab_hbm, emb_hbm, out_hbm,
             tab_v, ftab_v, xv, yv, zv,
             fb0, fb1, ib0, ib1, gb0, gb1, ov, sem0, sem1, osem):
    wid = lax.axis_index("s") * _NC + lax.axis_index("c")
    pltpu.sync_copy(tab_hbm, tab_v)
    pltpu.sync_copy(ftab_hbm, ftab_v)

    _HMASK = (1 << 19) - 1

    def phase1h(l, fb, ib):
        scale = ftab_v[pl.ds(l * 16, 16)]
        offv2 = tab_v[pl.ds(l * 96 + 64, 16)]
        for g in range(_CH // 16):
            s = pl.ds(g * 16, 16)
            px = xv[s] * scale + 0.5
            py = yv[s] * scale + 0.5
            pz = zv[s] * scale + 0.5
            gx = px.astype(jnp.int32)
            gy = py.astype(jnp.int32)
            gz = pz.astype(jnp.int32)
            fb[pl.ds(g * 16, 16)] = px - gx.astype(jnp.float32)
            fb[pl.ds(_CH + g * 16, 16)] = py - gy.astype(jnp.float32)
            fb[pl.ds(2 * _CH + g * 16, 16)] = pz - gz.astype(jnp.float32)
            x1 = gx + 1
            h1a = gy * _P1
            h1b = h1a + _P1
            h2a = gz * _P2
            h2b = h2a + _P2
            for c in range(8):
                s0 = x1 if (c & 1) else gx
                w = s0 ^ (h1b if (c & 2) else h1a) ^ (h2b if (c & 4) else h2a)
                a0 = ((w & _HMASK) * 2) - (w & 127) + offv2
                cb = c * 2 * _CH + g * 32
                ib[pl.ds(cb, 16)] = a0
                ib[pl.ds(cb + 16, 16)] = a0 + 128

    def phase1d(l, fb, ib):
        scale = float(2.0 ** l * 16.0 - 1.0)
        res = 16 << l
        mask = res ** 3 - 1
        off2 = 2 * sum((16 << j) ** 3 for j in range(l))
        for g in range(_CH // 16):
            s = pl.ds(g * 16, 16)
            px = xv[s] * scale + 0.5
            py = yv[s] * scale + 0.5
            pz = zv[s] * scale + 0.5
            gx = px.astype(jnp.int32)
            gy = py.astype(jnp.int32)
            gz = pz.astype(jnp.int32)
            fb[pl.ds(g * 16, 16)] = px - gx.astype(jnp.float32)
            fb[pl.ds(_CH + g * 16, 16)] = py - gy.astype(jnp.float32)
            fb[pl.ds(2 * _CH + g * 16, 16)] = pz - gz.astype(jnp.float32)
            x1 = gx + 1
            d1a = gy * res
            d1b = d1a + res
            d2a = gz * (res * res)
            d2b = d2a + (res * res)
            for c in range(8):
                s0 = x1 if (c & 1) else gx
                w = s0 + (d1b if (c & 2) else d1a) + (d2b if (c & 4) else d2a)
                a0 = ((w & mask) * 2) - (w & 127) + off2
                cb = c * 2 * _CH + g * 32
                ib[pl.ds(cb, 16)] = a0
                ib[pl.ds(cb + 16, 16)] = a0 + 128

    def fire(ib, gb, sem):
        pltpu.async_copy(emb_hbm.at[ib], gb, sem)

    def drain(ib, gb, sem):
        pltpu.make_async_copy(emb_hbm.at[ib], gb, sem).wait()

    def phase3(l, fb, gb):
        lvl_off = l * (2 * _CH)
        for g in range(_CH // 16):
            s = pl.ds(g * 16, 16)
            fx = fb[s]
            fy = fb[pl.ds(_CH + g * 16, 16)]
            fz = fb[pl.ds(2 * _CH + g * 16, 16)]
            for ch in range(2):
                v = [gb[pl.ds(c * 2 * _CH + g * 32 + ch * 16, 16)]
                     for c in range(8)]
                u0 = v[0] + fx * (v[1] - v[0])
                u1 = v[2] + fx * (v[3] - v[2])
                u2 = v[4] + fx * (v[5] - v[4])
                u3 = v[6] + fx * (v[7] - v[6])
                w0 = u0 + fy * (u1 - u0)
                w1 = u2 + fy * (u3 - u2)
                r = w0 + fz * (w1 - w0)
                ov[pl.ds(lvl_off + (ch * _CH + g * 16), 16)] = r

    def chunk_body(t, _):
        base = wid * _PTS + t * _CH
        pltpu.sync_copy(inp_hbm.at[pl.ds(base, _CH)], xv)
        pltpu.sync_copy(inp_hbm.at[pl.ds(_B + base, _CH)], yv)
        pltpu.sync_copy(inp_hbm.at[pl.ds(2 * _B + base, _CH)], zv)

        phase1d(0, fb0, ib0)
        fire(ib0, gb0, sem0)
        phase1d(1, fb1, ib1)
        fire(ib1, gb1, sem1)
        drain(ib0, gb0, sem0)
        phase3(jnp.int32(0), fb0, gb0)
        phase1d(2, fb0, ib0)
        fire(ib0, gb0, sem0)
        drain(ib1, gb1, sem1)
        phase3(jnp.int32(1), fb1, gb1)
        phase1h(jnp.int32(3), fb1, ib1)
        fire(ib1, gb1, sem1)
        drain(ib0, gb0, sem0)
        phase3(jnp.int32(2), fb0, gb0)

        def pair_body(k, _):
            la = 2 * k + 4
            phase1h(la, fb0, ib0)
            fire(ib0, gb0, sem0)
            drain(ib1, gb1, sem1)
            phase3(la - 1, fb1, gb1)
            phase1h(la + 1, fb1, ib1)
            fire(ib1, gb1, sem1)
            drain(ib0, gb0, sem0)
            phase3(la, fb0, gb0)

        lax.fori_loop(0, 6, pair_body, None)
        drain(ib1, gb1, sem1)
        phase3(jnp.int32(_L - 1), fb1, gb1)

        bi = wid * (_PTS // _CH) + t
        for jg in range(4):
            pltpu.async_copy(
                ov.at[pl.ds(jg * 1024, 1024)],
                out_hbm.at[pl.ds(jg * (_B * 8) + bi * 1024, 1024)], osem)
        for jg in range(4):
            pltpu.make_async_copy(
                ov.at[pl.ds(jg * 1024, 1024)],
                out_hbm.at[pl.ds(jg * (_B * 8) + bi * 1024, 1024)], osem).wait()

    lax.fori_loop(0, _NCHUNK, chunk_body, None)


_mesh = plsc.VectorSubcoreMesh(core_axis_name="c", subcore_axis_name="s",
                               num_cores=_NC, num_subcores=_NS)

_sc_call = pl.kernel(
    _sc_body,
    out_type=jax.ShapeDtypeStruct((_B * _L * _C,), jnp.float32),
    mesh=_mesh,
    scratch_types=[
        pltpu.VMEM((_L * 6 * 16,), jnp.int32),
        pltpu.VMEM((_L * 16,), jnp.float32),
        pltpu.VMEM((_CH,), jnp.float32),
        pltpu.VMEM((_CH,), jnp.float32),
        pltpu.VMEM((_CH,), jnp.float32),
        pltpu.VMEM((3 * _CH,), jnp.float32),
        pltpu.VMEM((3 * _CH,), jnp.float32),
        pltpu.VMEM((_GSZ,), jnp.int32),
        pltpu.VMEM((_GSZ,), jnp.int32),
        pltpu.VMEM((_GSZ,), jnp.float32),
        pltpu.VMEM((_GSZ,), jnp.float32),
        pltpu.VMEM((_CH * _L * _C,), jnp.float32),
        pltpu.SemaphoreType.DMA,
        pltpu.SemaphoreType.DMA,
        pltpu.SemaphoreType.DMA,
    ],
)


def kernel(inputs, embeddings):
    inp_t = inputs.T.reshape(-1)
    tab = jnp.asarray(_TAB)
    ftab = jnp.asarray(_FTAB)
    vemb = embeddings.reshape(-1, 128, _C).transpose(0, 2, 1).reshape(-1)
    out = _sc_call(inp_t, tab, ftab, vemb)
    out = out.reshape(4, _B // 128, 8, 128).transpose(1, 3, 0, 2)
    return out.reshape(_B, _L * _C)
